# baseline (speedup 1.0000x reference)
import functools
import numpy as np
import jax
import jax.numpy as jnp
from jax import lax
from jax.experimental import pallas as pl
from jax.experimental.pallas import tpu as pltpu
from jax.experimental.pallas import tpu_sc as plsc

_N, _LQ, _C, _L, _M, _P = 4, 2048, 256, 4, 8, 4
_D = _C // _M
_LENS = (2048, 1024, 512, 256)
_LV = sum(_LENS)
_STARTS = (0, 2048, 3072, 3584)
_LP = _L * _P
_QC = 128
_QB = 512

_T_LANE = np.tile(np.repeat(np.array(_LENS, np.float32), _P), _M).reshape(1, 128)
_S_LANE = np.tile(np.repeat(np.array(_STARTS, np.int32), _P), _M).reshape(1, 128)
_SEL = np.zeros((_L, 128), np.float32)
for _lane in range(128):
    _SEL[(_lane % _LP) // _P, _lane] = 1.0

_HPERM = np.concatenate([np.arange(0, _D, 2), np.arange(1, _D, 2)])
_CPERM = np.concatenate([m * _D + _HPERM for m in range(_M)])


_RB = _LV // (_LQ // _QB)


def _out_kernel(h_ref, w_ref, b_ref, o_ref):
    h2 = jnp.concatenate([h_ref[0, m] for m in range(_M)], axis=-1)
    o_ref[0] = jnp.dot(h2, w_ref[...], preferred_element_type=jnp.float32,
                       precision=lax.Precision.HIGHEST) + b_ref[...]


def _out_proj(heads, wT, b, rb):
    return pl.pallas_call(
        _out_kernel,
        grid=(_N, _LQ // rb),
        in_specs=[
            pl.BlockSpec((1, _M, rb, _D), lambda n, i: (n, 0, i, 0)),
            pl.BlockSpec((_C, _C), lambda n, i: (0, 0)),
            pl.BlockSpec((1, _C), lambda n, i: (0, 0)),
        ],
        out_specs=pl.BlockSpec((1, rb, _C), lambda n, i: (n, i, 0)),
        out_shape=jax.ShapeDtypeStruct((_N, _LQ, _C), jnp.float32),
    )(heads, wT, b.reshape(1, _C))


def _prep_kernel(xf_ref, wvT_ref, bv_ref, q_ref, rp_ref, woffT_ref, boff_ref,
                 wattnT_ref, battn_ref, tlane_ref, slane_ref, sel_ref,
                 val_ref, loc_ref, aw_ref, p0_ref, p1_ref):
    y = jnp.dot(xf_ref[0], wvT_ref[...], preferred_element_type=jnp.float32,
                precision=lax.Precision.HIGHEST) + bv_ref[...]
    yb = y.astype(jnp.bfloat16)
    for m in range(_M):
        val_ref[0, m] = yb[:, m * _D:(m + 1) * _D]

    q = q_ref[0]
    qb = q.shape[0]
    off = jnp.dot(q, woffT_ref[...], preferred_element_type=jnp.float32, precision=lax.Precision.HIGHEST) + boff_ref[...]
    a = jnp.dot(q, wattnT_ref[...], preferred_element_type=jnp.float32, precision=lax.Precision.HIGHEST) + battn_ref[...]
    a3 = a.reshape(qb, _M, _LP)
    amax = jnp.max(a3, axis=-1, keepdims=True)
    e = jnp.exp(a3 - amax)
    aw = (e / jnp.sum(e, axis=-1, keepdims=True)).reshape(qb, 128)
    aw_ref[0] = aw

    t_lane = tlane_ref[...]
    ref_lane = jnp.dot(rp_ref[0], sel_ref[...],
                       preferred_element_type=jnp.float32, precision=lax.Precision.HIGHEST)
    loc = ref_lane + off / t_lane
    loc_ref[0] = loc

    xg = 2.0 * loc - 1.0
    x = ((xg + 1.0) * t_lane - 1.0) * 0.5
    x0 = jnp.floor(x)
    w1 = x - x0
    w0 = 1.0 - w1
    x0i = x0.astype(jnp.int32)
    x1i = x0i + 1
    tl_i = t_lane.astype(jnp.int32)
    s_lane = slane_ref[...]
    v0 = ((x0i >= 0) & (x0i < tl_i)).astype(jnp.float32)
    v1 = ((x1i >= 0) & (x1i < tl_i)).astype(jnp.float32)
    r0 = jnp.clip(x0i, 0, tl_i - 1) + s_lane
    r1 = jnp.clip(x1i, 0, tl_i - 1) + s_lane
    ww0 = aw * w0 * v0
    ww1 = aw * w1 * v1
    b0 = (lax.bitcast_convert_type(ww0, jnp.int32) + 0x8000) >> 16
    b1 = (lax.bitcast_convert_type(ww1, jnp.int32) + 0x8000) >> 16
    p0 = (r0 << 16) | (b0 & 0xFFFF)
    p1 = (r1 << 16) | (b1 & 0xFFFF)
    for m in range(_M):
        p0_ref[0, m] = p0[:, m * _LP:(m + 1) * _LP]
        p1_ref[0, m] = p1[:, m * _LP:(m + 1) * _LP]


def _prep(input_flatten, WvT, bv, query, refp, WoffT, boff, WattnT, battn):
    f32 = jnp.float32
    out_shapes = [
        jax.ShapeDtypeStruct((_N, _M, _LV, _D), jnp.bfloat16),
        jax.ShapeDtypeStruct((_N, _LQ, 128), f32),
        jax.ShapeDtypeStruct((_N, _LQ, 128), f32),
        jax.ShapeDtypeStruct((_N, _M, _LQ, _LP), jnp.int32),
        jax.ShapeDtypeStruct((_N, _M, _LQ, _LP), jnp.int32),
    ]
    val_spec = pl.BlockSpec((1, _M, _RB, _D), lambda n, i: (n, 0, i, 0))
    vec_spec = pl.BlockSpec((1, _QB, 128), lambda n, i: (n, i, 0))
    hm_spec = pl.BlockSpec((1, _M, _QB, _LP), lambda n, i: (n, 0, i, 0))
    return pl.pallas_call(
        _prep_kernel,
        grid=(_N, _LQ // _QB),
        in_specs=[
            pl.BlockSpec((1, _RB, _C), lambda n, i: (n, i, 0)),
            pl.BlockSpec((_C, _C), lambda n, i: (0, 0)),
            pl.BlockSpec((1, _C), lambda n, i: (0, 0)),
            pl.BlockSpec((1, _QB, _C), lambda n, i: (n, i, 0)),
            pl.BlockSpec((1, _QB, _L), lambda n, i: (n, i, 0)),
            pl.BlockSpec((_C, 128), lambda n, i: (0, 0)),
            pl.BlockSpec((1, 128), lambda n, i: (0, 0)),
            pl.BlockSpec((_C, 128), lambda n, i: (0, 0)),
            pl.BlockSpec((1, 128), lambda n, i: (0, 0)),
            pl.BlockSpec((1, 128), lambda n, i: (0, 0)),
            pl.BlockSpec((1, 128), lambda n, i: (0, 0)),
            pl.BlockSpec((_L, 128), lambda n, i: (0, 0)),
        ],
        out_specs=[val_spec, vec_spec, vec_spec, hm_spec, hm_spec],
        out_shape=out_shapes,
    )(input_flatten, WvT, bv.reshape(1, _C), query, refp,
      WoffT, boff.reshape(1, 128), WattnT, battn.reshape(1, 128),
      jnp.asarray(_T_LANE), jnp.asarray(_S_LANE), jnp.asarray(_SEL))


def _sample_sc(value, p0, p1):
    mesh = plsc.VectorSubcoreMesh(core_axis_name="c", subcore_axis_name="s",
                                  num_cores=2, num_subcores=16)

    nch = _LQ // _QC

    @functools.partial(
        pl.kernel,
        mesh=mesh,
        compiler_params=pltpu.CompilerParams(use_tc_tiling_on_sc=False,
                                             needs_layout_passes=False),
        out_type=jax.ShapeDtypeStruct((_N, _M, _LQ, _D), jnp.float32),
        scratch_types=[
            pltpu.VMEM((_LV, _D), jnp.bfloat16),
            pltpu.VMEM((_QC, _LP), jnp.int32),
            pltpu.VMEM((_QC, _LP), jnp.int32),
            pltpu.VMEM((_QC, _LP), jnp.int32),
            pltpu.VMEM((_QC, _LP), jnp.int32),
            pltpu.VMEM((_QC, _D), jnp.float32),
            pltpu.SemaphoreType.DMA,
            pltpu.SemaphoreType.DMA,
        ],
    )
    def k(value_hbm, p0_hbm, p1_hbm, out_hbm, val_v,
          p0_a, p1_a, p0_b, p1_b, out_v, sem_a, sem_b):
        cid = lax.axis_index("c")
        sid = lax.axis_index("s")
        wid = sid * 2 + cid
        n = wid // _M
        m = wid % _M
        pltpu.sync_copy(value_hbm.at[n, m], val_v)

        def issue(ch, pb0, pb1, sem):
            q0 = ch * _QC
            pltpu.async_copy(p0_hbm.at[n, m, pl.ds(q0, _QC), :], pb0, sem)
            pltpu.async_copy(p1_hbm.at[n, m, pl.ds(q0, _QC), :], pb1, sem)

        def drain(ch, pb0, pb1, sem):
            q0 = ch * _QC
            pltpu.make_async_copy(
                p0_hbm.at[n, m, pl.ds(q0, _QC), :], pb0, sem).wait()
            pltpu.make_async_copy(
                p1_hbm.at[n, m, pl.ds(q0, _QC), :], pb1, sem).wait()

        def compute(ch, pb0, pb1):
            def q_body(qi, c2):
                for u in range(2):
                    qq = qi * 2 + u
                    acc0 = jnp.zeros((16,), jnp.float32)
                    acc1 = jnp.zeros((16,), jnp.float32)
                    pv0 = pb0[qq, pl.ds(0, _LP)]
                    pv1 = pb1[qq, pl.ds(0, _LP)]
                    for t in range(_LP):
                        s0 = pv0[t]
                        r0 = s0 >> 16
                        w0s = lax.bitcast_convert_type(s0 << 16, jnp.float32)
                        rv0 = val_v[r0, pl.ds(0, _D)]
                        e0, o0 = plsc.unpack(rv0,
                                             format=plsc.PackFormat.INTERLEAVED)
                        acc0 = acc0 + w0s * e0
                        acc1 = acc1 + w0s * o0
                        s1 = pv1[t]
                        r1 = s1 >> 16
                        w1s = lax.bitcast_convert_type(s1 << 16, jnp.float32)
                        rv1 = val_v[r1, pl.ds(0, _D)]
                        e1, o1 = plsc.unpack(rv1,
                                             format=plsc.PackFormat.INTERLEAVED)
                        acc0 = acc0 + w1s * e1
                        acc1 = acc1 + w1s * o1
                    out_v[qq, pl.ds(0, 16)] = acc0
                    out_v[qq, pl.ds(16, 16)] = acc1
                return c2

            lax.fori_loop(0, _QC // 2, q_body, 0)
            pltpu.sync_copy(out_v, out_hbm.at[n, m, pl.ds(ch * _QC, _QC), :])

        issue(0, p0_a, p1_a, sem_a)

        def pair_body(cc, carry):
            ch_a = cc * 2
            ch_b = ch_a + 1
            issue(ch_b, p0_b, p1_b, sem_b)
            drain(ch_a, p0_a, p1_a, sem_a)
            compute(ch_a, p0_a, p1_a)
            issue(jnp.minimum(ch_a + 2, nch - 1), p0_a, p1_a, sem_a)
            drain(ch_b, p0_b, p1_b, sem_b)
            compute(ch_b, p0_b, p1_b)
            return carry

        lax.fori_loop(0, nch // 2, pair_body, 0)
        drain(nch - 1, p0_a, p1_a, sem_a)

    return k(value, p0, p1)


def kernel(query, reference_points, input_flatten, input_temporal_lens,
           input_level_start_index, Wv, bv, Woff, boff, Wattn, battn,
           Wout, bout):
    n, lq, c = query.shape
    value, loc, aw, p0, p1 = _prep(
        input_flatten, Wv.T, bv,
        query, reference_points.reshape(n, lq, _L),
        Woff.T, boff, Wattn.T, battn,
    )

    heads = _sample_sc(value, p0, p1)
    out = _out_proj(heads, Wout.T[_CPERM], bout, 512)

    loc6 = loc.reshape(n, lq, _M, _L, _P, 1)
    sampling_locations = jnp.concatenate(
        [loc6, jnp.full_like(loc6, 0.5)], axis=-1)
    aw_out = aw.reshape(n, lq, _M, _L, _P)
    return out, sampling_locations, aw_out


# speedup vs baseline: 1.0809x; 1.0809x over previous
import functools
import numpy as np
import jax
import jax.numpy as jnp
from jax import lax
from jax.experimental import pallas as pl
from jax.experimental.pallas import tpu as pltpu
from jax.experimental.pallas import tpu_sc as plsc

_N, _LQ, _C, _L, _M, _P = 4, 2048, 256, 4, 8, 4
_D = _C // _M
_LENS = (2048, 1024, 512, 256)
_LV = sum(_LENS)
_STARTS = (0, 2048, 3072, 3584)
_LP = _L * _P
_QC = 64
_QB = 512

_T_LANE = np.tile(np.repeat(np.array(_LENS, np.float32), _P), _M).reshape(1, 128)
_S_LANE = np.tile(np.repeat(np.array(_STARTS, np.int32), _P), _M).reshape(1, 128)
_SEL = np.zeros((_L, 128), np.float32)
for _lane in range(128):
    _SEL[(_lane % _LP) // _P, _lane] = 1.0


_RB = _LV // (_LQ // _QB)


def _out_kernel(h_ref, w_ref, b_ref, o_ref):
    h2 = jnp.concatenate([h_ref[0, m] for m in range(_M)], axis=-1)
    o_ref[0] = jnp.dot(h2, w_ref[...], preferred_element_type=jnp.float32,
                       precision=lax.Precision.HIGHEST) + b_ref[...]


def _out_proj(heads, wT, b, rb):
    return pl.pallas_call(
        _out_kernel,
        grid=(_N, _LQ // rb),
        in_specs=[
            pl.BlockSpec((1, _M, rb, _D), lambda n, i: (n, 0, i, 0)),
            pl.BlockSpec((_C, _C), lambda n, i: (0, 0)),
            pl.BlockSpec((1, _C), lambda n, i: (0, 0)),
        ],
        out_specs=pl.BlockSpec((1, rb, _C), lambda n, i: (n, i, 0)),
        out_shape=jax.ShapeDtypeStruct((_N, _LQ, _C), jnp.float32),
    )(heads, wT, b.reshape(1, _C))


def _prep_kernel(xf_ref, wvT_ref, bv_ref, q_ref, rp_ref, woffT_ref, boff_ref,
                 wattnT_ref, battn_ref, tlane_ref, slane_ref, sel_ref,
                 val_ref, loc_ref, aw_ref, p0_ref, p1_ref):
    y = jnp.dot(xf_ref[0], wvT_ref[...], preferred_element_type=jnp.float32,
                precision=lax.Precision.HIGHEST) + bv_ref[...]
    for m in range(_M):
        val_ref[0, m] = y[:, m * _D:(m + 1) * _D]

    q = q_ref[0]
    qb = q.shape[0]
    off = jnp.dot(q, woffT_ref[...], preferred_element_type=jnp.float32, precision=lax.Precision.HIGHEST) + boff_ref[...]
    a = jnp.dot(q, wattnT_ref[...], preferred_element_type=jnp.float32, precision=lax.Precision.HIGHEST) + battn_ref[...]
    a3 = a.reshape(qb, _M, _LP)
    amax = jnp.max(a3, axis=-1, keepdims=True)
    e = jnp.exp(a3 - amax)
    aw = (e / jnp.sum(e, axis=-1, keepdims=True)).reshape(qb, 128)
    aw_ref[0] = aw

    t_lane = tlane_ref[...]
    ref_lane = jnp.dot(rp_ref[0], sel_ref[...],
                       preferred_element_type=jnp.float32, precision=lax.Precision.HIGHEST)
    loc = ref_lane + off / t_lane
    loc_ref[0] = loc

    xg = 2.0 * loc - 1.0
    x = ((xg + 1.0) * t_lane - 1.0) * 0.5
    x0 = jnp.floor(x)
    w1 = x - x0
    w0 = 1.0 - w1
    x0i = x0.astype(jnp.int32)
    x1i = x0i + 1
    tl_i = t_lane.astype(jnp.int32)
    s_lane = slane_ref[...]
    v0 = ((x0i >= 0) & (x0i < tl_i)).astype(jnp.float32)
    v1 = ((x1i >= 0) & (x1i < tl_i)).astype(jnp.float32)
    r0 = jnp.clip(x0i, 0, tl_i - 1) + s_lane
    r1 = jnp.clip(x1i, 0, tl_i - 1) + s_lane
    ww0 = aw * w0 * v0
    ww1 = aw * w1 * v1
    b0 = (lax.bitcast_convert_type(ww0, jnp.int32) + 0x8000) >> 16
    b1 = (lax.bitcast_convert_type(ww1, jnp.int32) + 0x8000) >> 16
    p0 = (r0 << 16) | (b0 & 0xFFFF)
    p1 = (r1 << 16) | (b1 & 0xFFFF)
    for m in range(_M):
        p0_ref[0, m] = p0[:, m * _LP:(m + 1) * _LP]
        p1_ref[0, m] = p1[:, m * _LP:(m + 1) * _LP]


def _prep(input_flatten, WvT, bv, query, refp, WoffT, boff, WattnT, battn):
    f32 = jnp.float32
    out_shapes = [
        jax.ShapeDtypeStruct((_N, _M, _LV, _D), f32),
        jax.ShapeDtypeStruct((_N, _LQ, 128), f32),
        jax.ShapeDtypeStruct((_N, _LQ, 128), f32),
        jax.ShapeDtypeStruct((_N, _M, _LQ, _LP), jnp.int32),
        jax.ShapeDtypeStruct((_N, _M, _LQ, _LP), jnp.int32),
    ]
    val_spec = pl.BlockSpec((1, _M, _RB, _D), lambda n, i: (n, 0, i, 0))
    vec_spec = pl.BlockSpec((1, _QB, 128), lambda n, i: (n, i, 0))
    hm_spec = pl.BlockSpec((1, _M, _QB, _LP), lambda n, i: (n, 0, i, 0))
    return pl.pallas_call(
        _prep_kernel,
        grid=(_N, _LQ // _QB),
        in_specs=[
            pl.BlockSpec((1, _RB, _C), lambda n, i: (n, i, 0)),
            pl.BlockSpec((_C, _C), lambda n, i: (0, 0)),
            pl.BlockSpec((1, _C), lambda n, i: (0, 0)),
            pl.BlockSpec((1, _QB, _C), lambda n, i: (n, i, 0)),
            pl.BlockSpec((1, _QB, _L), lambda n, i: (n, i, 0)),
            pl.BlockSpec((_C, 128), lambda n, i: (0, 0)),
            pl.BlockSpec((1, 128), lambda n, i: (0, 0)),
            pl.BlockSpec((_C, 128), lambda n, i: (0, 0)),
            pl.BlockSpec((1, 128), lambda n, i: (0, 0)),
            pl.BlockSpec((1, 128), lambda n, i: (0, 0)),
            pl.BlockSpec((1, 128), lambda n, i: (0, 0)),
            pl.BlockSpec((_L, 128), lambda n, i: (0, 0)),
        ],
        out_specs=[val_spec, vec_spec, vec_spec, hm_spec, hm_spec],
        out_shape=out_shapes,
    )(input_flatten, WvT, bv.reshape(1, _C), query, refp,
      WoffT, boff.reshape(1, 128), WattnT, battn.reshape(1, 128),
      jnp.asarray(_T_LANE), jnp.asarray(_S_LANE), jnp.asarray(_SEL))


def _sample_sc(value, p0, p1):
    mesh = plsc.VectorSubcoreMesh(core_axis_name="c", subcore_axis_name="s",
                                  num_cores=2, num_subcores=16)

    nch = _LQ // _QC

    @functools.partial(
        pl.kernel,
        mesh=mesh,
        compiler_params=pltpu.CompilerParams(use_tc_tiling_on_sc=False),
        out_type=jax.ShapeDtypeStruct((_N, _M, _LQ, _D), jnp.float32),
        scratch_types=[
            pltpu.VMEM((_LV, _D), jnp.float32),
            pltpu.VMEM((_QC, _LP), jnp.int32),
            pltpu.VMEM((_QC, _LP), jnp.int32),
            pltpu.VMEM((_QC, _LP), jnp.int32),
            pltpu.VMEM((_QC, _LP), jnp.int32),
            pltpu.VMEM((_QC, _D), jnp.float32),
            pltpu.SemaphoreType.DMA,
            pltpu.SemaphoreType.DMA,
        ],
    )
    def k(value_hbm, p0_hbm, p1_hbm, out_hbm, val_v,
          p0_a, p1_a, p0_b, p1_b, out_v, sem_a, sem_b):
        cid = lax.axis_index("c")
        sid = lax.axis_index("s")
        wid = sid * 2 + cid
        n = wid // _M
        m = wid % _M
        pltpu.sync_copy(value_hbm.at[n, m], val_v)

        def issue(ch, pb0, pb1, sem):
            q0 = ch * _QC
            pltpu.async_copy(p0_hbm.at[n, m, pl.ds(q0, _QC), :], pb0, sem)
            pltpu.async_copy(p1_hbm.at[n, m, pl.ds(q0, _QC), :], pb1, sem)

        def drain(ch, pb0, pb1, sem):
            q0 = ch * _QC
            pltpu.make_async_copy(
                p0_hbm.at[n, m, pl.ds(q0, _QC), :], pb0, sem).wait()
            pltpu.make_async_copy(
                p1_hbm.at[n, m, pl.ds(q0, _QC), :], pb1, sem).wait()

        def compute(ch, pb0, pb1):
            def q_body(qi, c2):
                for u in range(2):
                    qq = qi * 2 + u
                    acc0 = jnp.zeros((16,), jnp.float32)
                    acc1 = jnp.zeros((16,), jnp.float32)
                    pv0 = pb0[qq, pl.ds(0, _LP)]
                    pv1 = pb1[qq, pl.ds(0, _LP)]
                    for t in range(_LP):
                        s0 = pv0[t]
                        r0 = s0 >> 16
                        w0s = lax.bitcast_convert_type(s0 << 16, jnp.float32)
                        acc0 = acc0 + w0s * val_v[r0, pl.ds(0, 16)]
                        acc1 = acc1 + w0s * val_v[r0, pl.ds(16, 16)]
                        s1 = pv1[t]
                        r1 = s1 >> 16
                        w1s = lax.bitcast_convert_type(s1 << 16, jnp.float32)
                        acc0 = acc0 + w1s * val_v[r1, pl.ds(0, 16)]
                        acc1 = acc1 + w1s * val_v[r1, pl.ds(16, 16)]
                    out_v[qq, pl.ds(0, 16)] = acc0
                    out_v[qq, pl.ds(16, 16)] = acc1
                return c2

            lax.fori_loop(0, _QC // 2, q_body, 0)
            pltpu.sync_copy(out_v, out_hbm.at[n, m, pl.ds(ch * _QC, _QC), :])

        issue(0, p0_a, p1_a, sem_a)

        def pair_body(cc, carry):
            ch_a = cc * 2
            ch_b = ch_a + 1
            issue(ch_b, p0_b, p1_b, sem_b)
            drain(ch_a, p0_a, p1_a, sem_a)
            compute(ch_a, p0_a, p1_a)
            issue(jnp.minimum(ch_a + 2, nch - 1), p0_a, p1_a, sem_a)
            drain(ch_b, p0_b, p1_b, sem_b)
            compute(ch_b, p0_b, p1_b)
            return carry

        lax.fori_loop(0, nch // 2, pair_body, 0)
        drain(nch - 1, p0_a, p1_a, sem_a)

    return k(value, p0, p1)


def kernel(query, reference_points, input_flatten, input_temporal_lens,
           input_level_start_index, Wv, bv, Woff, boff, Wattn, battn,
           Wout, bout):
    n, lq, c = query.shape
    value, loc, aw, p0, p1 = _prep(
        input_flatten, Wv.T, bv,
        query, reference_points.reshape(n, lq, _L),
        Woff.T, boff, Wattn.T, battn,
    )

    heads = _sample_sc(value, p0, p1)
    out = _out_proj(heads, Wout.T, bout, 512)

    loc6 = loc.reshape(n, lq, _M, _L, _P, 1)
    sampling_locations = jnp.concatenate(
        [loc6, jnp.full_like(loc6, 0.5)], axis=-1)
    aw_out = aw.reshape(n, lq, _M, _L, _P)
    return out, sampling_locations, aw_out


# speedup vs baseline: 1.0835x; 1.0024x over previous
import functools
import numpy as np
import jax
import jax.numpy as jnp
from jax import lax
from jax.experimental import pallas as pl
from jax.experimental.pallas import tpu as pltpu
from jax.experimental.pallas import tpu_sc as plsc

_N, _LQ, _C, _L, _M, _P = 4, 2048, 256, 4, 8, 4
_D = _C // _M
_LENS = (2048, 1024, 512, 256)
_LV = sum(_LENS)
_STARTS = (0, 2048, 3072, 3584)
_LP = _L * _P
_QC = 64
_QB = 512

_T_LANE = np.tile(np.repeat(np.array(_LENS, np.float32), _P), _M).reshape(1, 128)
_S_LANE = np.tile(np.repeat(np.array(_STARTS, np.int32), _P), _M).reshape(1, 128)
_SEL = np.zeros((_L, 128), np.float32)
for _lane in range(128):
    _SEL[(_lane % _LP) // _P, _lane] = 1.0


_RB = _LV // (_LQ // _QB)


def _out_kernel(h_ref, w_ref, b_ref, o_ref):
    h2 = jnp.concatenate([h_ref[0, m] for m in range(_M)], axis=-1)
    o_ref[0] = jnp.dot(h2, w_ref[...], preferred_element_type=jnp.float32,
                       precision=lax.Precision.HIGHEST) + b_ref[...]


def _out_proj(heads, wT, b, rb):
    return pl.pallas_call(
        _out_kernel,
        grid=(_N, _LQ // rb),
        in_specs=[
            pl.BlockSpec((1, _M, rb, _D), lambda n, i: (n, 0, i, 0)),
            pl.BlockSpec((_C, _C), lambda n, i: (0, 0)),
            pl.BlockSpec((1, _C), lambda n, i: (0, 0)),
        ],
        out_specs=pl.BlockSpec((1, rb, _C), lambda n, i: (n, i, 0)),
        out_shape=jax.ShapeDtypeStruct((_N, _LQ, _C), jnp.float32),
    )(heads, wT, b.reshape(1, _C))


def _prep_kernel(xf_ref, wvT_ref, bv_ref, q_ref, rp_ref, woffT_ref, boff_ref,
                 wattnT_ref, battn_ref, tlane_ref, slane_ref, sel_ref,
                 val_ref, loc_ref, aw_ref, p0_ref, p1_ref):
    y = jnp.dot(xf_ref[0], wvT_ref[...], preferred_element_type=jnp.float32,
                precision=lax.Precision.HIGHEST) + bv_ref[...]
    for m in range(_M):
        val_ref[0, m] = y[:, m * _D:(m + 1) * _D]

    q = q_ref[0]
    qb = q.shape[0]
    off = jnp.dot(q, woffT_ref[...], preferred_element_type=jnp.float32, precision=lax.Precision.HIGHEST) + boff_ref[...]
    a = jnp.dot(q, wattnT_ref[...], preferred_element_type=jnp.float32, precision=lax.Precision.HIGHEST) + battn_ref[...]
    a3 = a.reshape(qb, _M, _LP)
    amax = jnp.max(a3, axis=-1, keepdims=True)
    e = jnp.exp(a3 - amax)
    rs = 1.0 / jnp.sum(e, axis=-1, keepdims=True)
    aw = (e * rs).reshape(qb, 128)
    aw_ref[0] = aw

    t_lane = tlane_ref[...]
    ref_lane = jnp.dot(rp_ref[0], sel_ref[...],
                       preferred_element_type=jnp.float32, precision=lax.Precision.HIGHEST)
    loc = ref_lane + off / t_lane
    loc_ref[0] = loc

    xg = 2.0 * loc - 1.0
    x = ((xg + 1.0) * t_lane - 1.0) * 0.5
    x0 = jnp.floor(x)
    w1 = x - x0
    w0 = 1.0 - w1
    x0i = x0.astype(jnp.int32)
    x1i = x0i + 1
    tl_i = t_lane.astype(jnp.int32)
    s_lane = slane_ref[...]
    v0 = ((x0i >= 0) & (x0i < tl_i)).astype(jnp.float32)
    v1 = ((x1i >= 0) & (x1i < tl_i)).astype(jnp.float32)
    r0 = jnp.clip(x0i, 0, tl_i - 1) + s_lane
    r1 = jnp.clip(x1i, 0, tl_i - 1) + s_lane
    ww0 = aw * w0 * v0
    ww1 = aw * w1 * v1
    b0 = (lax.bitcast_convert_type(ww0, jnp.int32) + 0x8000) >> 16
    b1 = (lax.bitcast_convert_type(ww1, jnp.int32) + 0x8000) >> 16
    p0 = (r0 << 16) | (b0 & 0xFFFF)
    p1 = (r1 << 16) | (b1 & 0xFFFF)
    for m in range(_M):
        p0_ref[0, m] = p0[:, m * _LP:(m + 1) * _LP]
        p1_ref[0, m] = p1[:, m * _LP:(m + 1) * _LP]


def _prep(input_flatten, WvT, bv, query, refp, WoffT, boff, WattnT, battn):
    f32 = jnp.float32
    out_shapes = [
        jax.ShapeDtypeStruct((_N, _M, _LV, _D), f32),
        jax.ShapeDtypeStruct((_N, _LQ, 128), f32),
        jax.ShapeDtypeStruct((_N, _LQ, 128), f32),
        jax.ShapeDtypeStruct((_N, _M, _LQ, _LP), jnp.int32),
        jax.ShapeDtypeStruct((_N, _M, _LQ, _LP), jnp.int32),
    ]
    val_spec = pl.BlockSpec((1, _M, _RB, _D), lambda n, i: (n, 0, i, 0))
    vec_spec = pl.BlockSpec((1, _QB, 128), lambda n, i: (n, i, 0))
    hm_spec = pl.BlockSpec((1, _M, _QB, _LP), lambda n, i: (n, 0, i, 0))
    return pl.pallas_call(
        _prep_kernel,
        grid=(_N, _LQ // _QB),
        in_specs=[
            pl.BlockSpec((1, _RB, _C), lambda n, i: (n, i, 0)),
            pl.BlockSpec((_C, _C), lambda n, i: (0, 0)),
            pl.BlockSpec((1, _C), lambda n, i: (0, 0)),
            pl.BlockSpec((1, _QB, _C), lambda n, i: (n, i, 0)),
            pl.BlockSpec((1, _QB, _L), lambda n, i: (n, i, 0)),
            pl.BlockSpec((_C, 128), lambda n, i: (0, 0)),
            pl.BlockSpec((1, 128), lambda n, i: (0, 0)),
            pl.BlockSpec((_C, 128), lambda n, i: (0, 0)),
            pl.BlockSpec((1, 128), lambda n, i: (0, 0)),
            pl.BlockSpec((1, 128), lambda n, i: (0, 0)),
            pl.BlockSpec((1, 128), lambda n, i: (0, 0)),
            pl.BlockSpec((_L, 128), lambda n, i: (0, 0)),
        ],
        out_specs=[val_spec, vec_spec, vec_spec, hm_spec, hm_spec],
        out_shape=out_shapes,
    )(input_flatten, WvT, bv.reshape(1, _C), query, refp,
      WoffT, boff.reshape(1, 128), WattnT, battn.reshape(1, 128),
      jnp.asarray(_T_LANE), jnp.asarray(_S_LANE), jnp.asarray(_SEL))


def _sample_sc(value, p0, p1):
    mesh = plsc.VectorSubcoreMesh(core_axis_name="c", subcore_axis_name="s",
                                  num_cores=2, num_subcores=16)

    nch = _LQ // _QC

    @functools.partial(
        pl.kernel,
        mesh=mesh,
        compiler_params=pltpu.CompilerParams(use_tc_tiling_on_sc=False),
        out_type=jax.ShapeDtypeStruct((_N, _M, _LQ, _D), jnp.float32),
        scratch_types=[
            pltpu.VMEM((_LV, _D), jnp.float32),
            pltpu.VMEM((_QC, _LP), jnp.int32),
            pltpu.VMEM((_QC, _LP), jnp.int32),
            pltpu.VMEM((_QC, _LP), jnp.int32),
            pltpu.VMEM((_QC, _LP), jnp.int32),
            pltpu.VMEM((_QC, _D), jnp.float32),
            pltpu.SemaphoreType.DMA,
            pltpu.SemaphoreType.DMA,
        ],
    )
    def k(value_hbm, p0_hbm, p1_hbm, out_hbm, val_v,
          p0_a, p1_a, p0_b, p1_b, out_v, sem_a, sem_b):
        cid = lax.axis_index("c")
        sid = lax.axis_index("s")
        wid = sid * 2 + cid
        n = wid // _M
        m = wid % _M
        pltpu.sync_copy(value_hbm.at[n, m], val_v)

        def issue(ch, pb0, pb1, sem):
            q0 = ch * _QC
            pltpu.async_copy(p0_hbm.at[n, m, pl.ds(q0, _QC), :], pb0, sem)
            pltpu.async_copy(p1_hbm.at[n, m, pl.ds(q0, _QC), :], pb1, sem)

        def drain(ch, pb0, pb1, sem):
            q0 = ch * _QC
            pltpu.make_async_copy(
                p0_hbm.at[n, m, pl.ds(q0, _QC), :], pb0, sem).wait()
            pltpu.make_async_copy(
                p1_hbm.at[n, m, pl.ds(q0, _QC), :], pb1, sem).wait()

        def compute(ch, pb0, pb1):
            def q_body(qi, c2):
                for u in range(2):
                    qq = qi * 2 + u
                    acc0 = jnp.zeros((16,), jnp.float32)
                    acc1 = jnp.zeros((16,), jnp.float32)
                    pv0 = pb0[qq, pl.ds(0, _LP)]
                    pv1 = pb1[qq, pl.ds(0, _LP)]
                    for t in range(_LP):
                        s0 = pv0[t]
                        r0 = s0 >> 16
                        w0s = lax.bitcast_convert_type(s0 << 16, jnp.float32)
                        acc0 = acc0 + w0s * val_v[r0, pl.ds(0, 16)]
                        acc1 = acc1 + w0s * val_v[r0, pl.ds(16, 16)]
                        s1 = pv1[t]
                        r1 = s1 >> 16
                        w1s = lax.bitcast_convert_type(s1 << 16, jnp.float32)
                        acc0 = acc0 + w1s * val_v[r1, pl.ds(0, 16)]
                        acc1 = acc1 + w1s * val_v[r1, pl.ds(16, 16)]
                    out_v[qq, pl.ds(0, 16)] = acc0
                    out_v[qq, pl.ds(16, 16)] = acc1
                return c2

            lax.fori_loop(0, _QC // 2, q_body, 0)
            pltpu.sync_copy(out_v, out_hbm.at[n, m, pl.ds(ch * _QC, _QC), :])

        issue(0, p0_a, p1_a, sem_a)

        def pair_body(cc, carry):
            ch_a = cc * 2
            ch_b = ch_a + 1
            issue(ch_b, p0_b, p1_b, sem_b)
            drain(ch_a, p0_a, p1_a, sem_a)
            compute(ch_a, p0_a, p1_a)
            issue(jnp.minimum(ch_a + 2, nch - 1), p0_a, p1_a, sem_a)
            drain(ch_b, p0_b, p1_b, sem_b)
            compute(ch_b, p0_b, p1_b)
            return carry

        lax.fori_loop(0, nch // 2, pair_body, 0)
        drain(nch - 1, p0_a, p1_a, sem_a)

    return k(value, p0, p1)


def kernel(query, reference_points, input_flatten, input_temporal_lens,
           input_level_start_index, Wv, bv, Woff, boff, Wattn, battn,
           Wout, bout):
    n, lq, c = query.shape
    value, loc, aw, p0, p1 = _prep(
        input_flatten, Wv.T, bv,
        query, reference_points.reshape(n, lq, _L),
        Woff.T, boff, Wattn.T, battn,
    )

    heads = _sample_sc(value, p0, p1)
    out = _out_proj(heads, Wout.T, bout, 512)

    loc6 = loc.reshape(n, lq, _M, _L, _P, 1)
    sampling_locations = jnp.concatenate(
        [loc6, jnp.full_like(loc6, 0.5)], axis=-1)
    aw_out = aw.reshape(n, lq, _M, _L, _P)
    return out, sampling_locations, aw_out
